# baseline (speedup 1.0000x reference)
import functools

import jax
import jax.numpy as jnp
from jax import lax
from jax.experimental import pallas as pl
from jax.experimental.pallas import tpu as pltpu
from jax.experimental.pallas import tpu_sc as plsc

_VOCAB = 1000000
_N_EMBD = 32
_B = 4096
_T = 200
_B_TOTAL = _B * _T
_NW = 32
_B_PER_W = _B_TOTAL // _NW
_CHUNK = 640
_NBUF = 4
_N_CHUNKS = _B_PER_W // _CHUNK
_NGRP = _N_CHUNKS // _NBUF

_TBLK = 8192


def _eye32():
    r = lax.broadcasted_iota(jnp.int32, (32, 32), 0)
    c = lax.broadcasted_iota(jnp.int32, (32, 32), 1)
    return jnp.where(r == c, 1.0, 0.0).astype(jnp.float32)


def _transpose_table_body(tt_ref, out_ref):
    out_ref[...] = lax.dot_general(
        tt_ref[...], _eye32(),
        dimension_numbers=(((0,), (0,)), ((), ())),
        preferred_element_type=jnp.float32,
        precision=lax.Precision.HIGHEST,
    )


def _transpose_table(tt):
    grid = (_VOCAB + _TBLK - 1) // _TBLK
    return pl.pallas_call(
        _transpose_table_body,
        grid=(grid,),
        in_specs=[pl.BlockSpec((32, _TBLK), lambda j: (0, j))],
        out_specs=pl.BlockSpec((_TBLK, 32), lambda j: (j, 0)),
        out_shape=jax.ShapeDtypeStruct((_VOCAB, _N_EMBD), jnp.float32),
    )(tt)


def _transpose_out_body(rows_ref, out_ref):
    out_ref[0, :, :] = lax.dot_general(
        _eye32(), rows_ref[...],
        dimension_numbers=(((1,), (1,)), ((), ())),
        preferred_element_type=jnp.float32,
        precision=lax.Precision.HIGHEST,
    )


def _transpose_out(rows):
    return pl.pallas_call(
        _transpose_out_body,
        grid=(_T,),
        in_specs=[pl.BlockSpec((_B, _N_EMBD), lambda t: (t, 0))],
        out_specs=pl.BlockSpec((1, _N_EMBD, _B), lambda t: (t, 0, 0)),
        out_shape=jax.ShapeDtypeStruct((_T, _N_EMBD, _B), jnp.float32),
    )(rows)


_mesh = plsc.VectorSubcoreMesh(core_axis_name="c", subcore_axis_name="s")


@functools.partial(
    pl.kernel,
    mesh=_mesh,
    out_type=jax.ShapeDtypeStruct((_B_TOTAL, _N_EMBD), jnp.float32),
    scratch_types=[
        pltpu.VMEM((_B_PER_W,), jnp.int32),
    ]
    + [pltpu.VMEM((_CHUNK, _N_EMBD), jnp.float32) for _ in range(_NBUF)]
    + [pltpu.SemaphoreType.DMA for _ in range(2 * _NBUF)],
    compiler_params=pltpu.CompilerParams(use_tc_tiling_on_sc=False),
)
def _gather_kernel(idx_hbm, table_hbm, out_hbm, idx_v, *bufs_and_sems):
    rows = bufs_and_sems[:_NBUF]
    gsem = bufs_and_sems[_NBUF:2 * _NBUF]
    wsem = bufs_and_sems[2 * _NBUF:]

    wid = lax.axis_index("s") * 2 + lax.axis_index("c")
    base = wid * _B_PER_W

    pltpu.sync_copy(idx_hbm.at[pl.ds(base, _B_PER_W)], idx_v)

    def gather_copy(i, b):
        return pltpu.make_async_copy(
            table_hbm.at[idx_v.at[pl.ds(i * _CHUNK, _CHUNK)]],
            rows[b],
            gsem[b],
        )

    def write_copy(i, b):
        return pltpu.make_async_copy(
            rows[b],
            out_hbm.at[pl.ds(base + i * _CHUNK, _CHUNK)],
            wsem[b],
        )

    for b in range(_NBUF):
        gather_copy(b, b).start()

    @pl.loop(0, _NGRP)
    def _grp(g):
        for b in range(_NBUF):
            i = g * _NBUF + b
            gather_copy(i, b).wait()
            write_copy(i, b).start()
        for b in range(_NBUF):
            i = g * _NBUF + b
            write_copy(i, b).wait()

            @pl.when(g < _NGRP - 1)
            def _():
                gather_copy(i + _NBUF, b).start()


def kernel(idx, table):
    lin = _transpose_table(table.T)
    flat_idx = jnp.swapaxes(idx, 0, 1).reshape(-1)
    rows = _gather_kernel(flat_idx, lin)
    out3 = _transpose_out(rows)
    return jnp.transpose(out3, (2, 0, 1))


# speedup vs baseline: 1.8577x; 1.8577x over previous
import functools

import jax
import jax.numpy as jnp
from jax import lax
from jax.experimental import pallas as pl
from jax.experimental.pallas import tpu as pltpu
from jax.experimental.pallas import tpu_sc as plsc

_VOCAB = 1000000
_N_EMBD = 32
_B = 4096
_T = 200
_B_TOTAL = _B * _T
_NW = 32
_B_PER_W = _B_TOTAL // _NW
_CHUNK = 640
_NBUF = 4
_N_CHUNKS = _B_PER_W // _CHUNK
_NGRP = _N_CHUNKS // _NBUF

_RB = 2048
_NJ = 123
_VPAD = 4 * _RB * _NJ


def _eye(n):
    r = lax.broadcasted_iota(jnp.int32, (n, n), 0)
    c = lax.broadcasted_iota(jnp.int32, (n, n), 1)
    return jnp.where(r == c, 1.0, 0.0).astype(jnp.float32)


def _transpose_table_body(a0, a1, a2, a3, out_ref):
    g = jnp.concatenate([a0[...], a1[...], a2[...], a3[...]], axis=0)
    out_ref[...] = lax.dot_general(
        g, _eye(128),
        dimension_numbers=(((0,), (0,)), ((), ())),
        preferred_element_type=jnp.float32,
        precision=lax.Precision.HIGHEST,
    )


def _transpose_table(tt):
    in_specs = [
        pl.BlockSpec(
            (32, _RB),
            functools.partial(
                lambda a, j: (0, jnp.minimum(4 * j + a, _VOCAB // _RB)), a
            ),
        )
        for a in range(4)
    ]
    return pl.pallas_call(
        _transpose_table_body,
        grid=(_NJ,),
        in_specs=in_specs,
        out_specs=pl.BlockSpec((_RB, 128), lambda j: (j, 0)),
        out_shape=jax.ShapeDtypeStruct((_VPAD // 4, 128), jnp.float32),
    )(tt, tt, tt, tt)


def _transpose_out_body(in_ref, out_ref):
    x = in_ref[...]
    z = jnp.concatenate([x[:, 32 * a:32 * (a + 1)] for a in range(4)], axis=0)
    out_ref[...] = lax.dot_general(
        _eye(32), z,
        dimension_numbers=(((1,), (1,)), ((), ())),
        preferred_element_type=jnp.float32,
        precision=lax.Precision.HIGHEST,
    )


def _transpose_out(rows128):
    return pl.pallas_call(
        _transpose_out_body,
        grid=(_T,),
        in_specs=[pl.BlockSpec((_B // 4, 128), lambda t: (t, 0))],
        out_specs=pl.BlockSpec((_N_EMBD, _B), lambda t: (t, 0)),
        out_shape=jax.ShapeDtypeStruct((_T * _N_EMBD, _B), jnp.float32),
    )(rows128)


_mesh = plsc.VectorSubcoreMesh(core_axis_name="c", subcore_axis_name="s")


@functools.partial(
    pl.kernel,
    mesh=_mesh,
    out_type=jax.ShapeDtypeStruct((_B_TOTAL, _N_EMBD), jnp.float32),
    scratch_types=[
        pltpu.VMEM((_B_PER_W,), jnp.int32),
    ]
    + [pltpu.VMEM((_CHUNK, _N_EMBD), jnp.float32) for _ in range(_NBUF)]
    + [pltpu.SemaphoreType.DMA for _ in range(2 * _NBUF)],
    compiler_params=pltpu.CompilerParams(use_tc_tiling_on_sc=False),
)
def _gather_kernel(idx_hbm, table_hbm, out_hbm, idx_v, *bufs_and_sems):
    rows = bufs_and_sems[:_NBUF]
    gsem = bufs_and_sems[_NBUF:2 * _NBUF]
    wsem = bufs_and_sems[2 * _NBUF:]

    wid = lax.axis_index("s") * 2 + lax.axis_index("c")
    base = wid * _B_PER_W

    pltpu.sync_copy(idx_hbm.at[pl.ds(base, _B_PER_W)], idx_v)

    def gather_copy(i, b):
        return pltpu.make_async_copy(
            table_hbm.at[idx_v.at[pl.ds(i * _CHUNK, _CHUNK)]],
            rows[b],
            gsem[b],
        )

    def write_copy(i, b):
        return pltpu.make_async_copy(
            rows[b],
            out_hbm.at[pl.ds(base + i * _CHUNK, _CHUNK)],
            wsem[b],
        )

    for b in range(_NBUF):
        gather_copy(b, b).start()

    @pl.loop(0, _NGRP)
    def _grp(g):
        for b in range(_NBUF):
            i = g * _NBUF + b
            gather_copy(i, b).wait()
            write_copy(i, b).start()
        for b in range(_NBUF):
            i = g * _NBUF + b
            write_copy(i, b).wait()

            @pl.when(g < _NGRP - 1)
            def _():
                gather_copy(i + _NBUF, b).start()


def kernel(idx, table):
    lin128 = _transpose_table(table.T)
    lin = lin128.reshape(_VPAD, _N_EMBD)
    v = jnp.swapaxes(idx, 0, 1).reshape(_T, 4, _B // 4)
    v = jnp.transpose(v, (0, 2, 1)).reshape(-1)
    flat_idx = ((v >> 13) << 13) + ((v & 2047) << 2) + ((v >> 11) & 3)
    rows = _gather_kernel(flat_idx, lin)
    out2 = _transpose_out(rows.reshape(_B_TOTAL * _N_EMBD // 128, 128))
    return jnp.transpose(out2.reshape(_T, _N_EMBD, _B), (2, 0, 1))


# speedup vs baseline: 2.5705x; 1.3837x over previous
import functools

import jax
import jax.numpy as jnp
from jax import lax
from jax.experimental import pallas as pl
from jax.experimental.pallas import tpu as pltpu
from jax.experimental.pallas import tpu_sc as plsc

_VOCAB = 1000000
_N_EMBD = 32
_B = 4096
_T = 200
_B_TOTAL = _B * _T
_NW = 32
_B_PER_W = _B_TOTAL // _NW
_CHUNK = 640
_NBUF = 4
_N_CHUNKS = _B_PER_W // _CHUNK
_NGRP = _N_CHUNKS // _NBUF

_RB = 2048
_NJ = 123
_VPAD = 4 * _RB * _NJ


def _eye(n):
    r = lax.broadcasted_iota(jnp.int32, (n, n), 0)
    c = lax.broadcasted_iota(jnp.int32, (n, n), 1)
    return jnp.where(r == c, 1.0, 0.0).astype(jnp.float32)


def _transpose_table_body(a0, a1, a2, a3, out_ref):
    g = jnp.concatenate([a0[...], a1[...], a2[...], a3[...]], axis=0)
    out_ref[...] = jnp.transpose(g)


def _transpose_table(tt):
    in_specs = [
        pl.BlockSpec(
            (32, _RB),
            functools.partial(
                lambda a, j: (0, jnp.minimum(4 * j + a, _VOCAB // _RB)), a
            ),
        )
        for a in range(4)
    ]
    return pl.pallas_call(
        _transpose_table_body,
        grid=(_NJ,),
        in_specs=in_specs,
        out_specs=pl.BlockSpec((_RB, 128), lambda j: (j, 0)),
        out_shape=jax.ShapeDtypeStruct((_VPAD // 4, 128), jnp.float32),
    )(tt, tt, tt, tt)


def _transpose_out_body(in_ref, out_ref):
    x = in_ref[...]
    z = jnp.concatenate([x[:, 32 * a:32 * (a + 1)] for a in range(4)], axis=0)
    out_ref[...] = jnp.transpose(z)


def _transpose_out(rows128):
    return pl.pallas_call(
        _transpose_out_body,
        grid=(_T,),
        in_specs=[pl.BlockSpec((_B // 4, 128), lambda t: (t, 0))],
        out_specs=pl.BlockSpec((_N_EMBD, _B), lambda t: (t, 0)),
        out_shape=jax.ShapeDtypeStruct((_T * _N_EMBD, _B), jnp.float32),
    )(rows128)


_mesh = plsc.VectorSubcoreMesh(core_axis_name="c", subcore_axis_name="s")


@functools.partial(
    pl.kernel,
    mesh=_mesh,
    out_type=jax.ShapeDtypeStruct((_B_TOTAL, _N_EMBD), jnp.float32),
    scratch_types=[
        pltpu.VMEM((_B_PER_W,), jnp.int32),
    ]
    + [pltpu.VMEM((_CHUNK, _N_EMBD), jnp.float32) for _ in range(_NBUF)]
    + [pltpu.SemaphoreType.DMA for _ in range(2 * _NBUF)],
    compiler_params=pltpu.CompilerParams(use_tc_tiling_on_sc=False),
)
def _gather_kernel(idx_hbm, table_hbm, out_hbm, idx_v, *bufs_and_sems):
    rows = bufs_and_sems[:_NBUF]
    gsem = bufs_and_sems[_NBUF:2 * _NBUF]
    wsem = bufs_and_sems[2 * _NBUF:]

    wid = lax.axis_index("s") * 2 + lax.axis_index("c")
    base = wid * _B_PER_W

    pltpu.sync_copy(idx_hbm.at[pl.ds(base, _B_PER_W)], idx_v)

    def gather_copy(i, b):
        return pltpu.make_async_copy(
            table_hbm.at[idx_v.at[pl.ds(i * _CHUNK, _CHUNK)]],
            rows[b],
            gsem[b],
        )

    def write_copy(i, b):
        return pltpu.make_async_copy(
            rows[b],
            out_hbm.at[pl.ds(base + i * _CHUNK, _CHUNK)],
            wsem[b],
        )

    for b in range(_NBUF):
        gather_copy(b, b).start()

    @pl.loop(0, _NGRP)
    def _grp(g):
        for b in range(_NBUF):
            i = g * _NBUF + b
            gather_copy(i, b).wait()
            write_copy(i, b).start()
        for b in range(_NBUF):
            i = g * _NBUF + b
            write_copy(i, b).wait()

            @pl.when(g < _NGRP - 1)
            def _():
                gather_copy(i + _NBUF, b).start()


def kernel(idx, table):
    lin128 = _transpose_table(table.T)
    lin = lin128.reshape(_VPAD, _N_EMBD)
    v = jnp.swapaxes(idx, 0, 1).reshape(_T, 4, _B // 4)
    v = jnp.transpose(v, (0, 2, 1)).reshape(-1)
    flat_idx = ((v >> 13) << 13) + ((v & 2047) << 2) + ((v >> 11) & 3)
    rows = _gather_kernel(flat_idx, lin)
    out2 = _transpose_out(rows.reshape(_B_TOTAL * _N_EMBD // 128, 128))
    return jnp.transpose(out2.reshape(_T, _N_EMBD, _B), (2, 0, 1))


# speedup vs baseline: 2.5716x; 1.0004x over previous
import functools

import jax
import jax.numpy as jnp
from jax import lax
from jax.experimental import pallas as pl
from jax.experimental.pallas import tpu as pltpu
from jax.experimental.pallas import tpu_sc as plsc

_VOCAB = 1000000
_N_EMBD = 32
_B = 4096
_T = 200
_B_TOTAL = _B * _T
_NW = 32
_B_PER_W = _B_TOTAL // _NW
_CHUNK = 640
_NBUF = 4
_N_CHUNKS = _B_PER_W // _CHUNK
_NGRP = _N_CHUNKS // _NBUF

_RB = 2048
_NJ = 123
_VPAD = 4 * _RB * _NJ


def _eye(n):
    r = lax.broadcasted_iota(jnp.int32, (n, n), 0)
    c = lax.broadcasted_iota(jnp.int32, (n, n), 1)
    return jnp.where(r == c, 1.0, 0.0).astype(jnp.float32)


def _transpose_table_body(a0, a1, a2, a3, out_ref):
    g = jnp.concatenate([a0[...], a1[...], a2[...], a3[...]], axis=0)
    out_ref[...] = jnp.transpose(g)


def _transpose_table(tt):
    in_specs = [
        pl.BlockSpec(
            (32, _RB),
            functools.partial(
                lambda a, j: (0, jnp.minimum(4 * j + a, _VOCAB // _RB)), a
            ),
        )
        for a in range(4)
    ]
    return pl.pallas_call(
        _transpose_table_body,
        grid=(_NJ,),
        in_specs=in_specs,
        out_specs=pl.BlockSpec((_RB, 128), lambda j: (j, 0)),
        out_shape=jax.ShapeDtypeStruct((_VPAD // 4, 128), jnp.float32),
    )(tt, tt, tt, tt)


def _transpose_out_body(in_ref, out_ref):
    x = in_ref[...]
    for a in range(4):
        out_ref[:, 1024 * a:1024 * (a + 1)] = jnp.transpose(
            x[:, 32 * a:32 * (a + 1)]
        )


def _transpose_out(rows128):
    return pl.pallas_call(
        _transpose_out_body,
        grid=(_T,),
        in_specs=[pl.BlockSpec((_B // 4, 128), lambda t: (t, 0))],
        out_specs=pl.BlockSpec((_N_EMBD, _B), lambda t: (t, 0)),
        out_shape=jax.ShapeDtypeStruct((_T * _N_EMBD, _B), jnp.float32),
    )(rows128)


_mesh = plsc.VectorSubcoreMesh(core_axis_name="c", subcore_axis_name="s")


@functools.partial(
    pl.kernel,
    mesh=_mesh,
    out_type=jax.ShapeDtypeStruct((_B_TOTAL, _N_EMBD), jnp.float32),
    scratch_types=[
        pltpu.VMEM((_B_PER_W,), jnp.int32),
    ]
    + [pltpu.VMEM((_CHUNK, _N_EMBD), jnp.float32) for _ in range(_NBUF)]
    + [pltpu.SemaphoreType.DMA for _ in range(2 * _NBUF)],
    compiler_params=pltpu.CompilerParams(use_tc_tiling_on_sc=False),
)
def _gather_kernel(idx_hbm, table_hbm, out_hbm, idx_v, *bufs_and_sems):
    rows = bufs_and_sems[:_NBUF]
    gsem = bufs_and_sems[_NBUF:2 * _NBUF]
    wsem = bufs_and_sems[2 * _NBUF:]

    wid = lax.axis_index("s") * 2 + lax.axis_index("c")
    base = wid * _B_PER_W

    pltpu.sync_copy(idx_hbm.at[pl.ds(base, _B_PER_W)], idx_v)

    def gather_copy(i, b):
        return pltpu.make_async_copy(
            table_hbm.at[idx_v.at[pl.ds(i * _CHUNK, _CHUNK)]],
            rows[b],
            gsem[b],
        )

    def write_copy(i, b):
        return pltpu.make_async_copy(
            rows[b],
            out_hbm.at[pl.ds(base + i * _CHUNK, _CHUNK)],
            wsem[b],
        )

    for b in range(_NBUF):
        gather_copy(b, b).start()

    @pl.loop(0, _NGRP)
    def _grp(g):
        for b in range(_NBUF):
            i = g * _NBUF + b
            gather_copy(i, b).wait()
            write_copy(i, b).start()
        for b in range(_NBUF):
            i = g * _NBUF + b
            write_copy(i, b).wait()

            @pl.when(g < _NGRP - 1)
            def _():
                gather_copy(i + _NBUF, b).start()


def kernel(idx, table):
    lin128 = _transpose_table(table.T)
    lin = lin128.reshape(_VPAD, _N_EMBD)
    v = jnp.swapaxes(idx, 0, 1).reshape(_T, 4, _B // 4)
    v = jnp.transpose(v, (0, 2, 1)).reshape(-1)
    flat_idx = ((v >> 13) << 13) + ((v & 2047) << 2) + ((v >> 11) & 3)
    rows = _gather_kernel(flat_idx, lin)
    out2 = _transpose_out(rows.reshape(_B_TOTAL * _N_EMBD // 128, 128))
    return jnp.transpose(out2.reshape(_T, _N_EMBD, _B), (2, 0, 1))
